# baseline (speedup 1.0000x reference)
import functools

import jax
import jax.numpy as jnp
from jax import lax
from jax.experimental import pallas as pl
from jax.experimental.pallas import tpu as pltpu
from jax.experimental.pallas import tpu_sc as plsc

ZDIM, YDIM, XDIM, CHANS = 256, 256, 256, 4
N = 1_000_000
NC, NS = 2, 16
NW = NC * NS
C = 1024
K = 31
NPAD = NW * C * K
JBLK = 128
NJ = C // JBLK

_OFFS = (0, 65536, 256, 65792, 1, 65537, 257, 65793)


def _body(table, zs, ys, xs, out,
          zv, yv, xv,
          i0, i1, i2, i3, i4, i5, i6, i7,
          dzv, dyv, dxv,
          r0, r1, r2, r3, r4, r5, r6, r7,
          outv, sem):
  idx_refs = (i0, i1, i2, i3, i4, i5, i6, i7)
  row_refs = (r0, r1, r2, r3, r4, r5, r6, r7)
  wid = lax.axis_index("s") * NC + lax.axis_index("c")
  lane = lax.iota(jnp.int32, 16)
  br = lane // 4
  cm = lane % 4
  scale = jnp.float32(255.0)
  one = jnp.float32(1.0)

  def chunk_body(t, carry):
    base = (wid * K + t) * C
    pltpu.sync_copy(zs.at[pl.ds(base, C)], zv)
    pltpu.sync_copy(ys.at[pl.ds(base, C)], yv)
    pltpu.sync_copy(xs.at[pl.ds(base, C)], xv)

    def idx_body(g, c2):
      o = g * 16
      z = zv[pl.ds(o, 16)] * scale
      y = yv[pl.ds(o, 16)] * scale
      x = xv[pl.ds(o, 16)] * scale
      zi = jnp.minimum(z.astype(jnp.int32), ZDIM - 2)
      yi = jnp.minimum(y.astype(jnp.int32), YDIM - 2)
      xi = jnp.minimum(x.astype(jnp.int32), XDIM - 2)
      dzv[pl.ds(o, 16)] = z - zi.astype(jnp.float32)
      dyv[pl.ds(o, 16)] = y - yi.astype(jnp.float32)
      dxv[pl.ds(o, 16)] = x - xi.astype(jnp.float32)
      r = (((zi << 8) | yi) << 8) | xi
      i0[pl.ds(o, 16)] = r
      i1[pl.ds(o, 16)] = r + _OFFS[1]
      i2[pl.ds(o, 16)] = r + _OFFS[2]
      i3[pl.ds(o, 16)] = r + _OFFS[3]
      i4[pl.ds(o, 16)] = r + _OFFS[4]
      i5[pl.ds(o, 16)] = r + _OFFS[5]
      i6[pl.ds(o, 16)] = r + _OFFS[6]
      i7[pl.ds(o, 16)] = r + _OFFS[7]
      return c2

    lax.fori_loop(0, C // 16, idx_body, 0)

    def gather_body(j, c2):
      o = j * JBLK
      descs = [
          pltpu.async_copy(table.at[ib.at[pl.ds(o, JBLK)]],
                           rb.at[pl.ds(o, JBLK)], sem)
          for ib, rb in zip(idx_refs, row_refs)
      ]
      for d in descs:
        d.wait()
      return c2

    lax.fori_loop(0, NJ, gather_body, 0)

    def comb_body(g, c2):
      rowi = g * 4 + br
      gz = plsc.load_gather(dzv, [rowi])
      gy = plsc.load_gather(dyv, [rowi])
      gx = plsc.load_gather(dxv, [rowi])
      uz = one - gz
      uy = one - gy
      ux = one - gx
      p00 = uy * ux
      p10 = gy * ux
      p01 = uy * gx
      p11 = gy * gx
      v0 = plsc.load_gather(r0, [rowi, cm])
      v1 = plsc.load_gather(r1, [rowi, cm])
      v2 = plsc.load_gather(r2, [rowi, cm])
      v3 = plsc.load_gather(r3, [rowi, cm])
      v4 = plsc.load_gather(r4, [rowi, cm])
      v5 = plsc.load_gather(r5, [rowi, cm])
      v6 = plsc.load_gather(r6, [rowi, cm])
      v7 = plsc.load_gather(r7, [rowi, cm])
      acc = (p00 * (uz * v0 + gz * v1) + p10 * (uz * v2 + gz * v3)
             + p01 * (uz * v4 + gz * v5) + p11 * (uz * v6 + gz * v7))
      outv[pl.ds(g * 16, 16)] = acc
      return c2

    lax.fori_loop(0, C // 4, comb_body, 0)

    pltpu.sync_copy(outv, out.at[pl.ds(base * CHANS, C * CHANS)])
    return carry

  lax.fori_loop(0, K, chunk_body, 0)


_field = functools.partial(
    pl.kernel,
    out_type=jax.ShapeDtypeStruct((NPAD * CHANS,), jnp.float32),
    compiler_params=pltpu.CompilerParams(
        use_tc_tiling_on_sc=False, needs_layout_passes=False),
    mesh=plsc.VectorSubcoreMesh(core_axis_name="c", subcore_axis_name="s"),
    scratch_types=[
        pltpu.VMEM((C,), jnp.float32),
        pltpu.VMEM((C,), jnp.float32),
        pltpu.VMEM((C,), jnp.float32),
    ] + [pltpu.VMEM((C,), jnp.int32) for _ in range(8)]
    + [
        pltpu.VMEM((C,), jnp.float32),
        pltpu.VMEM((C,), jnp.float32),
        pltpu.VMEM((C,), jnp.float32),
    ] + [pltpu.VMEM((C, CHANS), jnp.float32) for _ in range(8)]
    + [
        pltpu.VMEM((C * CHANS,), jnp.float32),
        pltpu.SemaphoreType.DMA,
    ],
)(_body)


def kernel(positions, data):
  pad = NPAD - N
  posp = jnp.concatenate(
      [positions, jnp.zeros((pad, 3), jnp.float32)], axis=0)
  zsc = posp[:, 0]
  ysc = posp[:, 1]
  xsc = posp[:, 2]
  table = data.reshape(ZDIM * YDIM * XDIM, CHANS)
  out_flat = _field(table, zsc, ysc, xsc)
  return out_flat.reshape(NPAD, CHANS)[:N]


# speedup vs baseline: 1.0042x; 1.0042x over previous
import functools

import jax
import jax.numpy as jnp
from jax import lax
from jax.experimental import pallas as pl
from jax.experimental.pallas import tpu as pltpu
from jax.experimental.pallas import tpu_sc as plsc

ZDIM, YDIM, XDIM, CHANS = 256, 256, 256, 4
N = 1_000_000
NC, NS = 2, 16
NW = NC * NS
C = 1024
K = 31
NPAD = NW * C * K
JBLK = 128
NJ = C // JBLK

_OFFS = (0, 65536, 256, 65792, 1, 65537, 257, 65793)


def _body(table, zs, ys, xs, out,
          zv, yv, xv,
          i0, i1, i2, i3, i4, i5, i6, i7,
          dzv, dyv, dxv,
          r0, r1, r2, r3, r4, r5, r6, r7,
          outv, sem):
  idx_refs = (i0, i1, i2, i3, i4, i5, i6, i7)
  row_refs = (r0, r1, r2, r3, r4, r5, r6, r7)
  wid = lax.axis_index("s") * NC + lax.axis_index("c")
  lane = lax.iota(jnp.int32, 16)
  br = lane // 4
  cm = lane % 4
  scale = jnp.float32(255.0)
  one = jnp.float32(1.0)

  def chunk_body(t, carry):
    base = (wid * K + t) * C
    pltpu.sync_copy(zs.at[pl.ds(base, C)], zv)
    pltpu.sync_copy(ys.at[pl.ds(base, C)], yv)
    pltpu.sync_copy(xs.at[pl.ds(base, C)], xv)

    def idx_body(g, c2):
      o = g * 16
      z = zv[pl.ds(o, 16)] * scale
      y = yv[pl.ds(o, 16)] * scale
      x = xv[pl.ds(o, 16)] * scale
      zi = jnp.minimum(z.astype(jnp.int32), ZDIM - 2)
      yi = jnp.minimum(y.astype(jnp.int32), YDIM - 2)
      xi = jnp.minimum(x.astype(jnp.int32), XDIM - 2)
      dzv[pl.ds(o, 16)] = z - zi.astype(jnp.float32)
      dyv[pl.ds(o, 16)] = y - yi.astype(jnp.float32)
      dxv[pl.ds(o, 16)] = x - xi.astype(jnp.float32)
      r = (((zi << 8) | yi) << 8) | xi
      i0[pl.ds(o, 16)] = r
      i1[pl.ds(o, 16)] = r + _OFFS[1]
      i2[pl.ds(o, 16)] = r + _OFFS[2]
      i3[pl.ds(o, 16)] = r + _OFFS[3]
      i4[pl.ds(o, 16)] = r + _OFFS[4]
      i5[pl.ds(o, 16)] = r + _OFFS[5]
      i6[pl.ds(o, 16)] = r + _OFFS[6]
      i7[pl.ds(o, 16)] = r + _OFFS[7]
      return c2

    lax.fori_loop(0, C // 16, idx_body, 0)

    descs = [
        pltpu.async_copy(table.at[ib.at[pl.ds(o, JBLK)]],
                         rb.at[pl.ds(o, JBLK)], sem)
        for o in range(0, C, JBLK)
        for ib, rb in zip(idx_refs, row_refs)
    ]
    for d in descs:
      d.wait()

    def comb_body(g, c2):
      rowi = g * 4 + br
      gz = plsc.load_gather(dzv, [rowi])
      gy = plsc.load_gather(dyv, [rowi])
      gx = plsc.load_gather(dxv, [rowi])
      uz = one - gz
      uy = one - gy
      ux = one - gx
      p00 = uy * ux
      p10 = gy * ux
      p01 = uy * gx
      p11 = gy * gx
      v0 = plsc.load_gather(r0, [rowi, cm])
      v1 = plsc.load_gather(r1, [rowi, cm])
      v2 = plsc.load_gather(r2, [rowi, cm])
      v3 = plsc.load_gather(r3, [rowi, cm])
      v4 = plsc.load_gather(r4, [rowi, cm])
      v5 = plsc.load_gather(r5, [rowi, cm])
      v6 = plsc.load_gather(r6, [rowi, cm])
      v7 = plsc.load_gather(r7, [rowi, cm])
      acc = (p00 * (uz * v0 + gz * v1) + p10 * (uz * v2 + gz * v3)
             + p01 * (uz * v4 + gz * v5) + p11 * (uz * v6 + gz * v7))
      outv[pl.ds(g * 16, 16)] = acc
      return c2

    lax.fori_loop(0, C // 4, comb_body, 0)

    pltpu.sync_copy(outv, out.at[pl.ds(base * CHANS, C * CHANS)])
    return carry

  lax.fori_loop(0, K, chunk_body, 0)


_field = functools.partial(
    pl.kernel,
    out_type=jax.ShapeDtypeStruct((NPAD * CHANS,), jnp.float32),
    compiler_params=pltpu.CompilerParams(
        use_tc_tiling_on_sc=False, needs_layout_passes=False),
    mesh=plsc.VectorSubcoreMesh(core_axis_name="c", subcore_axis_name="s"),
    scratch_types=[
        pltpu.VMEM((C,), jnp.float32),
        pltpu.VMEM((C,), jnp.float32),
        pltpu.VMEM((C,), jnp.float32),
    ] + [pltpu.VMEM((C,), jnp.int32) for _ in range(8)]
    + [
        pltpu.VMEM((C,), jnp.float32),
        pltpu.VMEM((C,), jnp.float32),
        pltpu.VMEM((C,), jnp.float32),
    ] + [pltpu.VMEM((C, CHANS), jnp.float32) for _ in range(8)]
    + [
        pltpu.VMEM((C * CHANS,), jnp.float32),
        pltpu.SemaphoreType.DMA,
    ],
)(_body)


def kernel(positions, data):
  pad = NPAD - N
  posp = jnp.concatenate(
      [positions, jnp.zeros((pad, 3), jnp.float32)], axis=0)
  zsc = posp[:, 0]
  ysc = posp[:, 1]
  xsc = posp[:, 2]
  table = data.reshape(ZDIM * YDIM * XDIM, CHANS)
  out_flat = _field(table, zsc, ysc, xsc)
  return out_flat.reshape(NPAD, CHANS)[:N]


# speedup vs baseline: 1.0046x; 1.0004x over previous
import functools

import jax
import jax.numpy as jnp
from jax import lax
from jax.experimental import pallas as pl
from jax.experimental.pallas import tpu as pltpu
from jax.experimental.pallas import tpu_sc as plsc

ZDIM, YDIM, XDIM, CHANS = 256, 256, 256, 4
N = 1_000_000
NC, NS = 2, 16
NW = NC * NS
C = 1024
K = 31
NPAD = NW * C * K
JBLK = 128
NJ = C // JBLK

_OFFS = (0, 65536, 256, 65792, 1, 65537, 257, 65793)


def _body(table, zs, ys, xs, out,
          zv, yv, xv,
          i0, i1, i2, i3, i4, i5, i6, i7,
          dzv, dyv, dxv,
          r0, r1, r2, r3, r4, r5, r6, r7,
          outv, s0, s1, s2, s3, s4, s5, s6, s7):
  idx_refs = (i0, i1, i2, i3, i4, i5, i6, i7)
  sems = (s0, s1, s2, s3, s4, s5, s6, s7)
  row_refs = (r0, r1, r2, r3, r4, r5, r6, r7)
  wid = lax.axis_index("s") * NC + lax.axis_index("c")
  lane = lax.iota(jnp.int32, 16)
  br = lane // 4
  cm = lane % 4
  scale = jnp.float32(255.0)
  one = jnp.float32(1.0)

  def chunk_body(t, carry):
    base = (wid * K + t) * C
    pltpu.sync_copy(zs.at[pl.ds(base, C)], zv)
    pltpu.sync_copy(ys.at[pl.ds(base, C)], yv)
    pltpu.sync_copy(xs.at[pl.ds(base, C)], xv)

    def idx_body(g, c2):
      o = g * 16
      z = zv[pl.ds(o, 16)] * scale
      y = yv[pl.ds(o, 16)] * scale
      x = xv[pl.ds(o, 16)] * scale
      zi = jnp.minimum(z.astype(jnp.int32), ZDIM - 2)
      yi = jnp.minimum(y.astype(jnp.int32), YDIM - 2)
      xi = jnp.minimum(x.astype(jnp.int32), XDIM - 2)
      dzv[pl.ds(o, 16)] = z - zi.astype(jnp.float32)
      dyv[pl.ds(o, 16)] = y - yi.astype(jnp.float32)
      dxv[pl.ds(o, 16)] = x - xi.astype(jnp.float32)
      r = (((zi << 8) | yi) << 8) | xi
      i0[pl.ds(o, 16)] = r
      i1[pl.ds(o, 16)] = r + _OFFS[1]
      i2[pl.ds(o, 16)] = r + _OFFS[2]
      i3[pl.ds(o, 16)] = r + _OFFS[3]
      i4[pl.ds(o, 16)] = r + _OFFS[4]
      i5[pl.ds(o, 16)] = r + _OFFS[5]
      i6[pl.ds(o, 16)] = r + _OFFS[6]
      i7[pl.ds(o, 16)] = r + _OFFS[7]
      return c2

    lax.fori_loop(0, C // 16, idx_body, 0)

    descs = [
        pltpu.async_copy(table.at[ib.at[pl.ds(o, JBLK)]],
                         rb.at[pl.ds(o, JBLK)], sm)
        for o in range(0, C, JBLK)
        for ib, rb, sm in zip(idx_refs, row_refs, sems)
    ]
    for d in descs:
      d.wait()

    def comb_body(g, c2):
      rowi = g * 4 + br
      gz = plsc.load_gather(dzv, [rowi])
      gy = plsc.load_gather(dyv, [rowi])
      gx = plsc.load_gather(dxv, [rowi])
      uz = one - gz
      uy = one - gy
      ux = one - gx
      p00 = uy * ux
      p10 = gy * ux
      p01 = uy * gx
      p11 = gy * gx
      v0 = plsc.load_gather(r0, [rowi, cm])
      v1 = plsc.load_gather(r1, [rowi, cm])
      v2 = plsc.load_gather(r2, [rowi, cm])
      v3 = plsc.load_gather(r3, [rowi, cm])
      v4 = plsc.load_gather(r4, [rowi, cm])
      v5 = plsc.load_gather(r5, [rowi, cm])
      v6 = plsc.load_gather(r6, [rowi, cm])
      v7 = plsc.load_gather(r7, [rowi, cm])
      acc = (p00 * (uz * v0 + gz * v1) + p10 * (uz * v2 + gz * v3)
             + p01 * (uz * v4 + gz * v5) + p11 * (uz * v6 + gz * v7))
      outv[pl.ds(g * 16, 16)] = acc
      return c2

    lax.fori_loop(0, C // 4, comb_body, 0)

    pltpu.sync_copy(outv, out.at[pl.ds(base * CHANS, C * CHANS)])
    return carry

  lax.fori_loop(0, K, chunk_body, 0)


_field = functools.partial(
    pl.kernel,
    out_type=jax.ShapeDtypeStruct((NPAD * CHANS,), jnp.float32),
    compiler_params=pltpu.CompilerParams(
        use_tc_tiling_on_sc=False, needs_layout_passes=False),
    mesh=plsc.VectorSubcoreMesh(core_axis_name="c", subcore_axis_name="s"),
    scratch_types=[
        pltpu.VMEM((C,), jnp.float32),
        pltpu.VMEM((C,), jnp.float32),
        pltpu.VMEM((C,), jnp.float32),
    ] + [pltpu.VMEM((C,), jnp.int32) for _ in range(8)]
    + [
        pltpu.VMEM((C,), jnp.float32),
        pltpu.VMEM((C,), jnp.float32),
        pltpu.VMEM((C,), jnp.float32),
    ] + [pltpu.VMEM((C, CHANS), jnp.float32) for _ in range(8)]
    + [
        pltpu.VMEM((C * CHANS,), jnp.float32),
    ] + [pltpu.SemaphoreType.DMA for _ in range(8)],
)(_body)


def kernel(positions, data):
  pad = NPAD - N
  posp = jnp.concatenate(
      [positions, jnp.zeros((pad, 3), jnp.float32)], axis=0)
  zsc = posp[:, 0]
  ysc = posp[:, 1]
  xsc = posp[:, 2]
  table = data.reshape(ZDIM * YDIM * XDIM, CHANS)
  out_flat = _field(table, zsc, ysc, xsc)
  return out_flat.reshape(NPAD, CHANS)[:N]


# speedup vs baseline: 1.0356x; 1.0308x over previous
import functools

import jax
import jax.numpy as jnp
from jax import lax
from jax.experimental import pallas as pl
from jax.experimental.pallas import tpu as pltpu
from jax.experimental.pallas import tpu_sc as plsc

ZDIM, YDIM, XDIM, CHANS = 256, 256, 256, 4
N = 1_000_000
NC, NS = 2, 16
NW = NC * NS
C = 1024
K = 31
NPAD = NW * C * K
JBLK = 128
TROW = 2 * CHANS
NROWS = ZDIM * YDIM * XDIM // 2

_ZY_OFFS = (0, 65536, 256, 65792)


def _body(table, pos, out,
          pv,
          a0, a1, a2, a3, b0, b1, b2, b3,
          dzv, dyv, dxv, parv,
          ra0, ra1, ra2, ra3, rb0, rb1, rb2, rb3,
          outv, s0, s1, s2, s3, s4, s5, s6, s7):
  idx_refs = (a0, a1, a2, a3, b0, b1, b2, b3)
  row_refs = (ra0, ra1, ra2, ra3, rb0, rb1, rb2, rb3)
  sems = (s0, s1, s2, s3, s4, s5, s6, s7)
  wid = lax.axis_index("s") * NC + lax.axis_index("c")
  lane = lax.iota(jnp.int32, 16)
  br = lane // 4
  cm = lane % 4
  lane3 = lane * 3
  scale = jnp.float32(255.0)
  one = jnp.float32(1.0)

  def chunk_body(t, carry):
    base = (wid * K + t) * C
    pltpu.sync_copy(pos.at[pl.ds(base * 3, C * 3)], pv)

    def idx_body(g, c2):
      o = g * 16
      p0 = g * 48 + lane3
      z = plsc.load_gather(pv, [p0]) * scale
      y = plsc.load_gather(pv, [p0 + 1]) * scale
      x = plsc.load_gather(pv, [p0 + 2]) * scale
      zi = jnp.minimum(z.astype(jnp.int32), ZDIM - 2)
      yi = jnp.minimum(y.astype(jnp.int32), YDIM - 2)
      xi = jnp.minimum(x.astype(jnp.int32), XDIM - 2)
      dzv[pl.ds(o, 16)] = z - zi.astype(jnp.float32)
      dyv[pl.ds(o, 16)] = y - yi.astype(jnp.float32)
      dxv[pl.ds(o, 16)] = x - xi.astype(jnp.float32)
      flat = (((zi << 8) | yi) << 8) | xi
      parv[pl.ds(o, 16)] = flat & 1
      fa0 = flat
      fa1 = flat + _ZY_OFFS[1]
      fa2 = flat + _ZY_OFFS[2]
      fa3 = flat + _ZY_OFFS[3]
      a0[pl.ds(o, 16)] = fa0 >> 1
      a1[pl.ds(o, 16)] = fa1 >> 1
      a2[pl.ds(o, 16)] = fa2 >> 1
      a3[pl.ds(o, 16)] = fa3 >> 1
      b0[pl.ds(o, 16)] = (fa0 + 1) >> 1
      b1[pl.ds(o, 16)] = (fa1 + 1) >> 1
      b2[pl.ds(o, 16)] = (fa2 + 1) >> 1
      b3[pl.ds(o, 16)] = (fa3 + 1) >> 1
      return c2

    lax.fori_loop(0, C // 16, idx_body, 0)

    descs = [
        pltpu.async_copy(table.at[ib.at[pl.ds(o, JBLK)]],
                         rb.at[pl.ds(o, JBLK)], sm)
        for o in range(0, C, JBLK)
        for ib, rb, sm in zip(idx_refs, row_refs, sems)
    ]
    for d in descs:
      d.wait()

    def comb_body(g, c2):
      rowi = g * 4 + br
      gz = plsc.load_gather(dzv, [rowi])
      gy = plsc.load_gather(dyv, [rowi])
      gx = plsc.load_gather(dxv, [rowi])
      pr = plsc.load_gather(parv, [rowi])
      ca = pr * 4 + cm
      cb = (4 - pr * 4) + cm
      uz = one - gz
      uy = one - gy
      w0 = uz * uy
      w1 = gz * uy
      w2 = uz * gy
      w3 = gz * gy
      va = (w0 * plsc.load_gather(ra0, [rowi, ca])
            + w1 * plsc.load_gather(ra1, [rowi, ca])
            + w2 * plsc.load_gather(ra2, [rowi, ca])
            + w3 * plsc.load_gather(ra3, [rowi, ca]))
      vb = (w0 * plsc.load_gather(rb0, [rowi, cb])
            + w1 * plsc.load_gather(rb1, [rowi, cb])
            + w2 * plsc.load_gather(rb2, [rowi, cb])
            + w3 * plsc.load_gather(rb3, [rowi, cb]))
      outv[pl.ds(g * 16, 16)] = va + gx * (vb - va)
      return c2

    lax.fori_loop(0, C // 4, comb_body, 0)

    pltpu.sync_copy(outv, out.at[pl.ds(base * CHANS, C * CHANS)])
    return carry

  lax.fori_loop(0, K, chunk_body, 0)


_field = functools.partial(
    pl.kernel,
    out_type=jax.ShapeDtypeStruct((NPAD * CHANS,), jnp.float32),
    compiler_params=pltpu.CompilerParams(
        use_tc_tiling_on_sc=False, needs_layout_passes=False),
    mesh=plsc.VectorSubcoreMesh(core_axis_name="c", subcore_axis_name="s"),
    scratch_types=[
        pltpu.VMEM((C * 3,), jnp.float32),
    ] + [pltpu.VMEM((C,), jnp.int32) for _ in range(8)]
    + [
        pltpu.VMEM((C,), jnp.float32),
        pltpu.VMEM((C,), jnp.float32),
        pltpu.VMEM((C,), jnp.float32),
        pltpu.VMEM((C,), jnp.int32),
    ] + [pltpu.VMEM((C, TROW), jnp.float32) for _ in range(8)]
    + [
        pltpu.VMEM((C * CHANS,), jnp.float32),
    ] + [pltpu.SemaphoreType.DMA for _ in range(8)],
)(_body)


def kernel(positions, data):
  pad = NPAD - N
  posp = jnp.concatenate(
      [positions, jnp.zeros((pad, 3), jnp.float32)], axis=0)
  table = data.reshape(NROWS, TROW)
  out_flat = _field(table, posp.reshape(NPAD * 3))
  return out_flat.reshape(NPAD, CHANS)[:N]


# speedup vs baseline: 4.6654x; 4.5052x over previous
import functools

import jax
import jax.numpy as jnp
from jax import lax
from jax.experimental import pallas as pl
from jax.experimental.pallas import tpu as pltpu
from jax.experimental.pallas import tpu_sc as plsc

ZDIM, YDIM, XDIM, CHANS = 256, 256, 256, 4
N = 1_000_000
NC, NS = 2, 16
NW = NC * NS
C = 1024
K = 31
NPAD = NW * C * K
JBLK = 128

_ZY_OFFS = (0, 1 << 18, 1 << 10, (1 << 18) + (1 << 10))


def _body(table, pos, out, *scr):
  pv = scr[0]
  idx_refs = scr[1:33]
  dzv, dyv, dxv = scr[33:36]
  val_refs = scr[36:44]
  outv = scr[44]
  sems = scr[45:53]
  wid = lax.axis_index("s") * NC + lax.axis_index("c")
  lane = lax.iota(jnp.int32, 16)
  br = lane // 4
  cm = lane % 4
  lane3 = lane * 3
  scale = jnp.float32(255.0)
  one = jnp.float32(1.0)

  def chunk_body(t, carry):
    base = (wid * K + t) * C
    pltpu.sync_copy(pos.at[pl.ds(base * 3, C * 3)], pv)

    def idx_body(g, c2):
      o = g * 16
      p0 = g * 48 + lane3
      z = plsc.load_gather(pv, [p0]) * scale
      y = plsc.load_gather(pv, [p0 + 1]) * scale
      x = plsc.load_gather(pv, [p0 + 2]) * scale
      zi = jnp.minimum(z.astype(jnp.int32), ZDIM - 2)
      yi = jnp.minimum(y.astype(jnp.int32), YDIM - 2)
      xi = jnp.minimum(x.astype(jnp.int32), XDIM - 2)
      dzv[pl.ds(o, 16)] = z - zi.astype(jnp.float32)
      dyv[pl.ds(o, 16)] = y - yi.astype(jnp.float32)
      dxv[pl.ds(o, 16)] = x - xi.astype(jnp.float32)
      x1 = xi + 1
      zy = (zi << 18) + (yi << 10)
      a0 = zy + ((xi >> 7) << 9) + (xi & 127)
      a1 = zy + ((x1 >> 7) << 9) + (x1 & 127)
      for k in range(4):
        for xsel, a in ((0, a0), (1, a1)):
          ak = a + _ZY_OFFS[k]
          for c in range(4):
            idx_refs[(k * 2 + xsel) * 4 + c][pl.ds(o, 16)] = ak + (c << 7)
      return c2

    lax.fori_loop(0, C // 16, idx_body, 0)

    descs = [
        pltpu.async_copy(
            table.at[idx_refs[s * 4 + c].at[pl.ds(o, JBLK)]],
            val_refs[s].at[c, pl.ds(o, JBLK)], sems[s])
        for o in range(0, C, JBLK)
        for s in range(8)
        for c in range(4)
    ]
    for d in descs:
      d.wait()

    def comb_body(g, c2):
      rowi = g * 4 + br
      gz = plsc.load_gather(dzv, [rowi])
      gy = plsc.load_gather(dyv, [rowi])
      gx = plsc.load_gather(dxv, [rowi])
      uz = one - gz
      uy = one - gy
      w0 = uz * uy
      w1 = gz * uy
      w2 = uz * gy
      w3 = gz * gy
      va = (w0 * plsc.load_gather(val_refs[0], [cm, rowi])
            + w1 * plsc.load_gather(val_refs[2], [cm, rowi])
            + w2 * plsc.load_gather(val_refs[4], [cm, rowi])
            + w3 * plsc.load_gather(val_refs[6], [cm, rowi]))
      vb = (w0 * plsc.load_gather(val_refs[1], [cm, rowi])
            + w1 * plsc.load_gather(val_refs[3], [cm, rowi])
            + w2 * plsc.load_gather(val_refs[5], [cm, rowi])
            + w3 * plsc.load_gather(val_refs[7], [cm, rowi]))
      outv[pl.ds(g * 16, 16)] = va + gx * (vb - va)
      return c2

    lax.fori_loop(0, C // 4, comb_body, 0)

    pltpu.sync_copy(outv, out.at[pl.ds(base * CHANS, C * CHANS)])
    return carry

  lax.fori_loop(0, K, chunk_body, 0)


_field = functools.partial(
    pl.kernel,
    out_type=jax.ShapeDtypeStruct((NPAD * CHANS,), jnp.float32),
    compiler_params=pltpu.CompilerParams(
        use_tc_tiling_on_sc=False, needs_layout_passes=False),
    mesh=plsc.VectorSubcoreMesh(core_axis_name="c", subcore_axis_name="s"),
    scratch_types=[
        pltpu.VMEM((C * 3,), jnp.float32),
    ] + [pltpu.VMEM((C,), jnp.int32) for _ in range(32)]
    + [
        pltpu.VMEM((C,), jnp.float32),
        pltpu.VMEM((C,), jnp.float32),
        pltpu.VMEM((C,), jnp.float32),
    ] + [pltpu.VMEM((4, C), jnp.float32) for _ in range(8)]
    + [
        pltpu.VMEM((C * CHANS,), jnp.float32),
    ] + [pltpu.SemaphoreType.DMA for _ in range(8)],
)(_body)


def kernel(positions, data):
  pad = NPAD - N
  posp = jnp.concatenate(
      [positions, jnp.zeros((pad, 3), jnp.float32)], axis=0)
  table = (data.reshape(ZDIM, YDIM, 2, 128, CHANS)
           .transpose(0, 1, 2, 4, 3)
           .reshape(ZDIM * YDIM * XDIM * CHANS))
  out_flat = _field(table, posp.reshape(NPAD * 3))
  return out_flat.reshape(NPAD, CHANS)[:N]


# speedup vs baseline: 10.8622x; 2.3282x over previous
import functools

import jax
import jax.numpy as jnp
from jax import lax
from jax.experimental import pallas as pl
from jax.experimental.pallas import tpu as pltpu
from jax.experimental.pallas import tpu_sc as plsc

ZDIM, YDIM, XDIM, CHANS = 256, 256, 256, 4
N = 1_000_000
NC, NS = 2, 16
NW = NC * NS
C = 1024
K = 31
NPAD = NW * C * K
JBLK = 128

_ZY_OFFS = (0, 1 << 18, 1 << 10, (1 << 18) + (1 << 10))


def _body(table, af0, af1, dzs, dys, dxs, out, *scr):
  a0v, a1v = scr[0:2]
  idx_refs = scr[2:34]
  dzv, dyv, dxv = scr[34:37]
  val_refs = scr[37:45]
  outv = scr[45]
  sems = scr[46:54]
  wid = lax.axis_index("s") * NC + lax.axis_index("c")
  lane = lax.iota(jnp.int32, 16)
  br = lane // 4
  cm = lane % 4
  one = jnp.float32(1.0)

  def chunk_body(t, carry):
    base = (wid * K + t) * C
    pltpu.sync_copy(af0.at[pl.ds(base, C)], a0v)
    pltpu.sync_copy(af1.at[pl.ds(base, C)], a1v)
    pltpu.sync_copy(dzs.at[pl.ds(base, C)], dzv)
    pltpu.sync_copy(dys.at[pl.ds(base, C)], dyv)
    pltpu.sync_copy(dxs.at[pl.ds(base, C)], dxv)

    def idx_body(g, c2):
      o = g * 16
      a0 = a0v[pl.ds(o, 16)]
      a1 = a1v[pl.ds(o, 16)]
      for k in range(4):
        for xsel, a in ((0, a0), (1, a1)):
          ak = a + _ZY_OFFS[k]
          for c in range(4):
            idx_refs[(k * 2 + xsel) * 4 + c][pl.ds(o, 16)] = ak + (c << 7)
      return c2

    lax.fori_loop(0, C // 16, idx_body, 0)

    descs = [
        pltpu.async_copy(
            table.at[idx_refs[s * 4 + c].at[pl.ds(o, JBLK)]],
            val_refs[s].at[c, pl.ds(o, JBLK)], sems[s])
        for o in range(0, C, JBLK)
        for s in range(8)
        for c in range(4)
    ]
    for d in descs:
      d.wait()

    def comb_body(g, c2):
      rowi = g * 4 + br
      gz = plsc.load_gather(dzv, [rowi])
      gy = plsc.load_gather(dyv, [rowi])
      gx = plsc.load_gather(dxv, [rowi])
      uz = one - gz
      uy = one - gy
      w0 = uz * uy
      w1 = gz * uy
      w2 = uz * gy
      w3 = gz * gy
      va = (w0 * plsc.load_gather(val_refs[0], [cm, rowi])
            + w1 * plsc.load_gather(val_refs[2], [cm, rowi])
            + w2 * plsc.load_gather(val_refs[4], [cm, rowi])
            + w3 * plsc.load_gather(val_refs[6], [cm, rowi]))
      vb = (w0 * plsc.load_gather(val_refs[1], [cm, rowi])
            + w1 * plsc.load_gather(val_refs[3], [cm, rowi])
            + w2 * plsc.load_gather(val_refs[5], [cm, rowi])
            + w3 * plsc.load_gather(val_refs[7], [cm, rowi]))
      outv[pl.ds(g * 16, 16)] = va + gx * (vb - va)
      return c2

    lax.fori_loop(0, C // 4, comb_body, 0)

    pltpu.sync_copy(outv, out.at[pl.ds(base * CHANS, C * CHANS)])
    return carry

  lax.fori_loop(0, K, chunk_body, 0)


_field = functools.partial(
    pl.kernel,
    out_type=jax.ShapeDtypeStruct((NPAD * CHANS,), jnp.float32),
    compiler_params=pltpu.CompilerParams(
        use_tc_tiling_on_sc=False, needs_layout_passes=False),
    mesh=plsc.VectorSubcoreMesh(core_axis_name="c", subcore_axis_name="s"),
    scratch_types=[
        pltpu.VMEM((C,), jnp.int32),
        pltpu.VMEM((C,), jnp.int32),
    ] + [pltpu.VMEM((C,), jnp.int32) for _ in range(32)]
    + [
        pltpu.VMEM((C,), jnp.float32),
        pltpu.VMEM((C,), jnp.float32),
        pltpu.VMEM((C,), jnp.float32),
    ] + [pltpu.VMEM((4, C), jnp.float32) for _ in range(8)]
    + [
        pltpu.VMEM((C * CHANS,), jnp.float32),
    ] + [pltpu.SemaphoreType.DMA for _ in range(8)],
)(_body)


def kernel(positions, data):
  pad = NPAD - N
  p = positions * jnp.float32(255.0)
  pi = jnp.minimum(p.astype(jnp.int32), ZDIM - 2)
  d = p - pi.astype(jnp.float32)
  zi, yi, xi = pi[:, 0], pi[:, 1], pi[:, 2]
  x1 = xi + 1
  zy = (zi << 18) + (yi << 10)
  af0 = zy + ((xi >> 7) << 9) + (xi & 127)
  af1 = zy + ((x1 >> 7) << 9) + (x1 & 127)
  pz = lambda v: jnp.pad(v, (0, pad))
  table = (data.reshape(ZDIM, YDIM, 2, 128, CHANS)
           .transpose(0, 1, 2, 4, 3)
           .reshape(ZDIM * YDIM * XDIM * CHANS))
  out_flat = _field(table, pz(af0), pz(af1),
                    pz(d[:, 0]), pz(d[:, 1]), pz(d[:, 2]))
  return out_flat.reshape(NPAD, CHANS)[:N]


# speedup vs baseline: 10.9628x; 1.0093x over previous
import functools

import jax
import jax.numpy as jnp
from jax import lax
from jax.experimental import pallas as pl
from jax.experimental.pallas import tpu as pltpu
from jax.experimental.pallas import tpu_sc as plsc

ZDIM, YDIM, XDIM, CHANS = 256, 256, 256, 4
N = 1_000_000
NC, NS = 2, 16
NW = NC * NS
C = 1024
K = 31
NPAD = NW * C * K
JBLK = 128

_ZY_OFFS = (0, 1 << 18, 1 << 10, (1 << 18) + (1 << 10))


def _body(table, af0, af1, dzs, dys, dxs, out, *scr):
  a0v, a1v = scr[0:2]
  idx_refs = scr[2:34]
  dzv, dyv, dxv = scr[34:37]
  val_refs = scr[37:45]
  outv = scr[45]
  sems = scr[46:62]
  wid = lax.axis_index("s") * NC + lax.axis_index("c")
  lane = lax.iota(jnp.int32, 16)
  br = lane // 4
  cm = lane % 4
  one = jnp.float32(1.0)

  def chunk_body(t, carry):
    base = (wid * K + t) * C
    pltpu.sync_copy(af0.at[pl.ds(base, C)], a0v)
    pltpu.sync_copy(af1.at[pl.ds(base, C)], a1v)
    pltpu.sync_copy(dzs.at[pl.ds(base, C)], dzv)
    pltpu.sync_copy(dys.at[pl.ds(base, C)], dyv)
    pltpu.sync_copy(dxs.at[pl.ds(base, C)], dxv)

    def idx_body(g, c2):
      o = g * 16
      a0 = a0v[pl.ds(o, 16)]
      a1 = a1v[pl.ds(o, 16)]
      for k in range(4):
        for xsel, a in ((0, a0), (1, a1)):
          ak = a + _ZY_OFFS[k]
          for c in range(4):
            idx_refs[(k * 2 + xsel) * 4 + c][pl.ds(o, 16)] = ak + (c << 7)
      return c2

    lax.fori_loop(0, C // 16, idx_body, 0)

    half = C // 2

    def fire(h):
      return [
          pltpu.async_copy(
              table.at[idx_refs[s * 4 + c].at[pl.ds(h * half + o, JBLK)]],
              val_refs[s].at[c, pl.ds(h * half + o, JBLK)], sems[h * 8 + s])
          for o in range(0, half, JBLK)
          for s in range(8)
          for c in range(4)
      ]

    descs0 = fire(0)
    descs1 = fire(1)
    for d in descs0:
      d.wait()

    def comb_body(g, c2):
      rowi = g * 4 + br
      gz = plsc.load_gather(dzv, [rowi])
      gy = plsc.load_gather(dyv, [rowi])
      gx = plsc.load_gather(dxv, [rowi])
      uz = one - gz
      uy = one - gy
      w0 = uz * uy
      w1 = gz * uy
      w2 = uz * gy
      w3 = gz * gy
      va = (w0 * plsc.load_gather(val_refs[0], [cm, rowi])
            + w1 * plsc.load_gather(val_refs[2], [cm, rowi])
            + w2 * plsc.load_gather(val_refs[4], [cm, rowi])
            + w3 * plsc.load_gather(val_refs[6], [cm, rowi]))
      vb = (w0 * plsc.load_gather(val_refs[1], [cm, rowi])
            + w1 * plsc.load_gather(val_refs[3], [cm, rowi])
            + w2 * plsc.load_gather(val_refs[5], [cm, rowi])
            + w3 * plsc.load_gather(val_refs[7], [cm, rowi]))
      outv[pl.ds(g * 16, 16)] = va + gx * (vb - va)
      return c2

    lax.fori_loop(0, C // 8, comb_body, 0)
    for d in descs1:
      d.wait()
    lax.fori_loop(C // 8, C // 4, comb_body, 0)

    pltpu.sync_copy(outv, out.at[pl.ds(base * CHANS, C * CHANS)])
    return carry

  lax.fori_loop(0, K, chunk_body, 0)


_field = functools.partial(
    pl.kernel,
    out_type=jax.ShapeDtypeStruct((NPAD * CHANS,), jnp.float32),
    compiler_params=pltpu.CompilerParams(
        use_tc_tiling_on_sc=False, needs_layout_passes=False),
    mesh=plsc.VectorSubcoreMesh(core_axis_name="c", subcore_axis_name="s"),
    scratch_types=[
        pltpu.VMEM((C,), jnp.int32),
        pltpu.VMEM((C,), jnp.int32),
    ] + [pltpu.VMEM((C,), jnp.int32) for _ in range(32)]
    + [
        pltpu.VMEM((C,), jnp.float32),
        pltpu.VMEM((C,), jnp.float32),
        pltpu.VMEM((C,), jnp.float32),
    ] + [pltpu.VMEM((4, C), jnp.float32) for _ in range(8)]
    + [
        pltpu.VMEM((C * CHANS,), jnp.float32),
    ] + [pltpu.SemaphoreType.DMA for _ in range(16)],
)(_body)


def kernel(positions, data):
  pad = NPAD - N
  p = positions * jnp.float32(255.0)
  pi = jnp.minimum(p.astype(jnp.int32), ZDIM - 2)
  d = p - pi.astype(jnp.float32)
  zi, yi, xi = pi[:, 0], pi[:, 1], pi[:, 2]
  x1 = xi + 1
  zy = (zi << 18) + (yi << 10)
  af0 = zy + ((xi >> 7) << 9) + (xi & 127)
  af1 = zy + ((x1 >> 7) << 9) + (x1 & 127)
  pz = lambda v: jnp.pad(v, (0, pad))
  table = (data.reshape(ZDIM, YDIM, 2, 128, CHANS)
           .transpose(0, 1, 2, 4, 3)
           .reshape(ZDIM * YDIM * XDIM * CHANS))
  out_flat = _field(table, pz(af0), pz(af1),
                    pz(d[:, 0]), pz(d[:, 1]), pz(d[:, 2]))
  return out_flat.reshape(NPAD, CHANS)[:N]
